# baseline (speedup 1.0000x reference)
import functools

import jax
import jax.numpy as jnp
from jax import lax
from jax.experimental import pallas as pl
from jax.experimental.pallas import tpu as pltpu
from jax.experimental.pallas import tpu_sc as plsc

_B, _N = 4096, 20000
_SLABS = 2
_SB = _B // _SLABS
_CHUNKS = _N // 16
_NC, _NS = 2, 16
_ROWS_W = _SB // (_NC * _NS)
_RB = 128
_NEG = float("-inf")
_Z16 = (16,)


def _sc_body(vv, m1_hbm, m2_hbm, idx_hbm, buf0, buf1, m1b, m2b, idxb, sem0, sem1):
    wid = lax.axis_index("s") * _NC + lax.axis_index("c")
    base = wid * _ROWS_W
    lane = lax.broadcasted_iota(jnp.int32, _Z16, 0)

    def copy_in(row, buf, sem):
        return pltpu.make_async_copy(vv.at[row], buf, sem)

    def reduce_row(buf, r):
        def step(i, c):
            m1, m2, idxv = c
            v = buf[pl.ds(i * 16, 16)]
            gt = v > m1
            m2 = jnp.maximum(m2, jnp.minimum(m1, v))
            m1 = jnp.maximum(m1, v)
            idxv = jnp.where(gt, lane + i * 16, idxv)
            return m1, m2, idxv

        init = (
            jnp.full(_Z16, _NEG, jnp.float32),
            jnp.full(_Z16, _NEG, jnp.float32),
            jnp.zeros(_Z16, jnp.int32),
        )
        m1, m2, idxv = lax.fori_loop(0, _CHUNKS, step, init, unroll=4)
        m1b[r, :] = m1
        m2b[r, :] = m2
        idxb[r, :] = idxv

    copy_in(base, buf0, sem0).start()

    def outer(r2, carry):
        row = base + r2 * 2
        copy_in(row + 1, buf1, sem1).start()
        copy_in(row, buf0, sem0).wait()
        reduce_row(buf0, r2 * 2)

        @pl.when(r2 < _ROWS_W // 2 - 1)
        def _():
            copy_in(row + 2, buf0, sem0).start()

        copy_in(row + 1, buf1, sem1).wait()
        reduce_row(buf1, r2 * 2 + 1)
        return carry

    lax.fori_loop(0, _ROWS_W // 2, outer, 0)
    pltpu.sync_copy(m1b, m1_hbm.at[pl.ds(base, _ROWS_W)])
    pltpu.sync_copy(m2b, m2_hbm.at[pl.ds(base, _ROWS_W)])
    pltpu.sync_copy(idxb, idx_hbm.at[pl.ds(base, _ROWS_W)])


_sc_reduce = functools.partial(
    pl.kernel,
    out_type=[
        jax.ShapeDtypeStruct((_SB, 16), jnp.float32),
        jax.ShapeDtypeStruct((_SB, 16), jnp.float32),
        jax.ShapeDtypeStruct((_SB, 16), jnp.int32),
    ],
    mesh=plsc.VectorSubcoreMesh(
        core_axis_name="c", subcore_axis_name="s", num_cores=_NC, num_subcores=_NS
    ),
    scratch_types=[
        pltpu.VMEM((_N,), jnp.float32),
        pltpu.VMEM((_N,), jnp.float32),
        pltpu.VMEM((_ROWS_W, 16), jnp.float32),
        pltpu.VMEM((_ROWS_W, 16), jnp.float32),
        pltpu.VMEM((_ROWS_W, 16), jnp.int32),
        pltpu.SemaphoreType.DMA,
        pltpu.SemaphoreType.DMA,
    ],
)(_sc_body)


def _finish(m1, m2, idxv):
    gmax = jnp.max(m1, axis=1, keepdims=True)
    cand = jnp.where(m1 == gmax, idxv, jnp.int32(2**30))
    gidx = jnp.min(cand, axis=1, keepdims=True)
    second = jnp.max(jnp.where(cand == gidx, m2, m1), axis=1, keepdims=True)
    pay = jnp.maximum(second, 0.0)
    return gidx, pay


def _expand_first_body(m1_ref, m2_ref, idx_ref, alloc_ref, pay_out_ref):
    gidx, pay = _finish(m1_ref[...], m2_ref[...], idx_ref[...])
    col = lax.broadcasted_iota(jnp.int32, (_RB, _N), 1)
    is_arg = col == gidx
    alloc_ref[...] = is_arg.astype(jnp.float32)
    pay_out_ref[...] = jnp.where(is_arg, pay, 0.0)


def _expand_alias_body(m1_ref, m2_ref, idx_ref, a_in, p_in, alloc_ref, pay_out_ref):
    del a_in, p_in
    _expand_first_body(m1_ref, m2_ref, idx_ref, alloc_ref, pay_out_ref)


def kernel(virtual_values):
    out_shape = jax.ShapeDtypeStruct((_B, _N), jnp.float32)
    in_spec = pl.BlockSpec((_RB, 16), lambda i: (i, 0))
    small_spec = pl.BlockSpec((8, 128), lambda i: (0, 0))
    alloc = pay = None
    for s in range(_SLABS):
        m1, m2, idx = _sc_reduce(
            lax.slice_in_dim(virtual_values, s * _SB, (s + 1) * _SB, axis=0)
        )
        out_spec = pl.BlockSpec(
            (_RB, _N), functools.partial(lambda s_, i: (s_ * (_SB // _RB) + i, 0), s)
        )
        if s == 0:
            alloc, pay = pl.pallas_call(
                _expand_first_body,
                grid=(_SB // _RB,),
                in_specs=[in_spec, in_spec, in_spec],
                out_specs=[out_spec, out_spec],
                out_shape=[out_shape, out_shape],
            )(m1, m2, idx)
        else:
            alloc, pay = pl.pallas_call(
                _expand_alias_body,
                grid=(_SB // _RB,),
                in_specs=[in_spec, in_spec, in_spec, small_spec, small_spec],
                out_specs=[out_spec, out_spec],
                out_shape=[out_shape, out_shape],
                input_output_aliases={3: 0, 4: 1},
            )(m1, m2, idx, alloc, pay)
    return (alloc, pay)


# speedup vs baseline: 1.0943x; 1.0943x over previous
import functools

import jax
import jax.numpy as jnp
from jax import lax
from jax.experimental import pallas as pl
from jax.experimental.pallas import tpu as pltpu
from jax.experimental.pallas import tpu_sc as plsc

_B, _N = 4096, 20000
_CHUNKS = _N // 16
_NC, _NS = 2, 16
_ROWS_W = _B // (_NC * _NS)
_NEG = float("-inf")
_Z16 = (16,)


def _sc_body(
    vv,
    alloc_hbm,
    pay_hbm,
    in0,
    in1,
    za0,
    za1,
    zp0,
    zp1,
    ff,
    fi,
    sem_in0,
    sem_in1,
    sem_a0,
    sem_a1,
    sem_p0,
    sem_p1,
):
    wid = lax.axis_index("s") * _NC + lax.axis_index("c")
    base = wid * _ROWS_W
    lane = lax.broadcasted_iota(jnp.int32, _Z16, 0)
    zeros16 = jnp.zeros(_Z16, jnp.float32)

    def copy_in(row, buf, sem):
        return pltpu.make_async_copy(vv.at[row], buf, sem)

    def copy_out(buf, out, row, sem):
        return pltpu.make_async_copy(buf, out.at[row], sem)

    def zinit(i, carry):
        za0[pl.ds(i * 16, 16)] = zeros16
        za1[pl.ds(i * 16, 16)] = zeros16
        zp0[pl.ds(i * 16, 16)] = zeros16
        zp1[pl.ds(i * 16, 16)] = zeros16
        return carry

    lax.fori_loop(0, _CHUNKS, zinit, 0)

    ff[pl.ds(16, 16)] = jnp.full(_Z16, _NEG, jnp.float32)
    fi[pl.ds(16, 16)] = jnp.full(_Z16, 2**30, jnp.int32)

    def redmax_f(vec):
        m = vec
        for s in (8, 4, 2, 1):
            ff[pl.ds(0, 16)] = m
            m = jnp.maximum(m, ff[pl.ds(s, 16)])
        return m[0]

    def redmin_i(vec):
        m = vec
        for s in (8, 4, 2, 1):
            fi[pl.ds(0, 16)] = m
            m = jnp.minimum(m, fi[pl.ds(s, 16)])
        return m[0]

    def reduce_row(buf):
        def step(i, c):
            m1, m2, idxv = c
            v = buf[pl.ds(i * 16, 16)]
            gt = v > m1
            m2 = jnp.maximum(m2, jnp.minimum(m1, v))
            m1 = jnp.maximum(m1, v)
            idxv = jnp.where(gt, lane + i * 16, idxv)
            return m1, m2, idxv

        init = (
            jnp.full(_Z16, _NEG, jnp.float32),
            jnp.full(_Z16, _NEG, jnp.float32),
            jnp.zeros(_Z16, jnp.int32),
        )
        m1, m2, idxv = lax.fori_loop(0, _CHUNKS, step, init, unroll=4)
        gmax = redmax_f(m1)
        cand = jnp.where(m1 == gmax, idxv, jnp.int32(2**30))
        gidx = redmin_i(cand)
        cand2 = jnp.where(idxv == gidx, m2, m1)
        second = redmax_f(cand2)
        return gidx, jnp.maximum(second, 0.0)

    def do_row(row, inbuf, insem, zba, zbp, sema, semp, pw, first):
        copy_in(row, inbuf, insem).wait()
        gidx, pay = reduce_row(inbuf)
        w16 = (gidx // 16) * 16
        off = gidx - w16

        @pl.when(jnp.logical_not(first))
        def _():
            copy_out(zba, alloc_hbm, row, sema).wait()
            copy_out(zbp, pay_hbm, row, semp).wait()

        zba[pl.ds(pw, 16)] = zeros16
        zbp[pl.ds(pw, 16)] = zeros16
        zba[pl.ds(w16, 16)] = jnp.where(lane == off, 1.0, 0.0)
        zbp[pl.ds(w16, 16)] = jnp.where(lane == off, pay, 0.0)
        copy_out(zba, alloc_hbm, row, sema).start()
        copy_out(zbp, pay_hbm, row, semp).start()
        return w16

    copy_in(base, in0, sem_in0).start()
    copy_in(base + 1, in1, sem_in1).start()

    def outer(r2, carry):
        pw0, pw1 = carry
        row = base + r2 * 2
        pw0 = do_row(row, in0, sem_in0, za0, zp0, sem_a0, sem_p0, pw0, r2 == 0)

        @pl.when(r2 < _ROWS_W // 2 - 1)
        def _():
            copy_in(row + 2, in0, sem_in0).start()

        pw1 = do_row(row + 1, in1, sem_in1, za1, zp1, sem_a1, sem_p1, pw1, r2 == 0)

        @pl.when(r2 < _ROWS_W // 2 - 1)
        def _():
            copy_in(row + 3, in1, sem_in1).start()

        return (pw0, pw1)

    lax.fori_loop(0, _ROWS_W // 2, outer, (jnp.int32(0), jnp.int32(0)))
    copy_out(za0, alloc_hbm, base, sem_a0).wait()
    copy_out(zp0, pay_hbm, base, sem_p0).wait()
    copy_out(za1, alloc_hbm, base, sem_a1).wait()
    copy_out(zp1, pay_hbm, base, sem_p1).wait()


_sc_auction = functools.partial(
    pl.kernel,
    out_type=[
        jax.ShapeDtypeStruct((_B, _N), jnp.float32),
        jax.ShapeDtypeStruct((_B, _N), jnp.float32),
    ],
    mesh=plsc.VectorSubcoreMesh(
        core_axis_name="c", subcore_axis_name="s", num_cores=_NC, num_subcores=_NS
    ),
    scratch_types=[
        pltpu.VMEM((_N,), jnp.float32),
        pltpu.VMEM((_N,), jnp.float32),
        pltpu.VMEM((_N,), jnp.float32),
        pltpu.VMEM((_N,), jnp.float32),
        pltpu.VMEM((_N,), jnp.float32),
        pltpu.VMEM((_N,), jnp.float32),
        pltpu.VMEM((32,), jnp.float32),
        pltpu.VMEM((32,), jnp.int32),
        pltpu.SemaphoreType.DMA,
        pltpu.SemaphoreType.DMA,
        pltpu.SemaphoreType.DMA,
        pltpu.SemaphoreType.DMA,
        pltpu.SemaphoreType.DMA,
        pltpu.SemaphoreType.DMA,
    ],
)(_sc_body)


def kernel(virtual_values):
    alloc, payments = _sc_auction(virtual_values)
    return (alloc, payments)
